# baseline (speedup 1.0000x reference)
import jax
import jax.numpy as jnp
from jax import lax
from jax.experimental import pallas as pl
from jax.experimental.pallas import tpu as pltpu
from jax.experimental.pallas import tpu_sc as plsc

B = 16384
L = 20
D = 64
NV = D // 16
NC = 2
NS = 16
LANES = 16
NW = NC * NS
PER_W = B // NW
C = 16
NCH = PER_W // C
IDS = C * L
WIDS = PER_W * L


def _body(ix_hbm, iy_hbm, tab_hbm, out_hbm,
          ixa, iya, rxv0, rxv1, ryv0, ryv1,
          stage, oacc, sem0, sem1):
    wid = lax.axis_index("s") * NC + lax.axis_index("c")
    lane = lax.iota(jnp.int32, LANES)
    rxv = (rxv0, rxv1)
    ryv = (ryv0, ryv1)
    sems = (sem0, sem1)

    pltpu.sync_copy(ix_hbm.at[pl.ds(wid * WIDS, WIDS)], ixa)
    pltpu.sync_copy(iy_hbm.at[pl.ds(wid * WIDS, WIDS)], iya)

    def fire(ch, b):
        pltpu.async_copy(tab_hbm.at[ixa.at[pl.ds(ch * IDS, IDS)]], rxv[b], sems[b])
        pltpu.async_copy(tab_hbm.at[iya.at[pl.ds(ch * IDS, IDS)]], ryv[b], sems[b])

    def drain(b):
        pltpu.make_async_copy(tab_hbm.at[pl.ds(0, IDS), :], rxv[b], sems[b]).wait()
        pltpu.make_async_copy(tab_hbm.at[pl.ds(0, IDS), :], ryv[b], sems[b]).wait()

    def step(ch, b):
        nxt = ch + 1

        @pl.when(nxt < NCH)
        def _():
            fire(nxt, 1 - b)

        drain(b)
        rx, ry = rxv[b], ryv[b]

        def row(r, rcarry):
            base = r * L
            ax = [rx[base, pl.ds(v * LANES, LANES)] for v in range(NV)]
            ay = [ry[base, pl.ds(v * LANES, LANES)] for v in range(NV)]
            for l in range(1, L):
                for v in range(NV):
                    ax[v] = ax[v] + rx[base + l, pl.ds(v * LANES, LANES)]
                    ay[v] = ay[v] + ry[base + l, pl.ds(v * LANES, LANES)]
            d01 = ax[0] * ay[0] + ax[1] * ay[1]
            d23 = ax[2] * ay[2] + ax[3] * ay[3]
            stage[r, :] = d01 + d23
            return rcarry

        lax.fori_loop(0, C, row, 0)

        dot = plsc.load_gather(stage, [lane, lax.broadcast(0, (LANES,))])
        for j in range(1, LANES):
            dot = dot + plsc.load_gather(stage, [lane, lax.broadcast(j, (LANES,))])
        y = 1.0 / (1.0 + jnp.exp(-dot))
        oacc[pl.ds(ch * C, C)] = y

    fire(0, 0)

    def pair(i, carry):
        step(2 * i, 0)
        step(2 * i + 1, 1)
        return carry

    lax.fori_loop(0, NCH // 2, pair, 0)
    pltpu.sync_copy(oacc, out_hbm.at[pl.ds(wid * PER_W, PER_W)])


def kernel(idx_x, idx_y, table):
    ix = idx_x.reshape(-1).astype(jnp.int32)
    iy = idx_y.reshape(-1).astype(jnp.int32)
    mesh = plsc.VectorSubcoreMesh(core_axis_name="c", subcore_axis_name="s")
    f = pl.kernel(
        _body,
        out_type=jax.ShapeDtypeStruct((B,), jnp.float32),
        mesh=mesh,
        compiler_params=pltpu.CompilerParams(
            needs_layout_passes=False, use_tc_tiling_on_sc=False),
        scratch_types=[
            pltpu.VMEM((WIDS,), jnp.int32),
            pltpu.VMEM((WIDS,), jnp.int32),
            pltpu.VMEM((IDS, D), jnp.float32),
            pltpu.VMEM((IDS, D), jnp.float32),
            pltpu.VMEM((IDS, D), jnp.float32),
            pltpu.VMEM((IDS, D), jnp.float32),
            pltpu.VMEM((C, LANES), jnp.float32),
            pltpu.VMEM((PER_W,), jnp.float32),
            pltpu.SemaphoreType.DMA,
            pltpu.SemaphoreType.DMA,
        ],
    )
    return f(ix, iy, table)
